# baseline (speedup 1.0000x reference)
import jax
import jax.numpy as jnp
from jax.experimental import pallas as pl

N_STAGES = 4
K = 1024
D = 256
M_BLK = 512


def _select_chosen(residual, rn, cn, codes):
    dots = jax.lax.dot_general(
        residual, codes, (((1,), (1,)), ((), ())),
        preferred_element_type=jnp.float32)
    dists = (rn + cn) - 2.0 * dots
    dmin = jnp.min(dists, axis=1, keepdims=True)
    col = jax.lax.broadcasted_iota(jnp.int32, dists.shape, 1)
    idx = jnp.min(jnp.where(dists == dmin, col, K), axis=1)
    onehot = (col == idx[:, None]).astype(jnp.bfloat16)
    c_hi = codes.astype(jnp.bfloat16)
    r1 = codes - c_hi.astype(jnp.float32)
    c_mid = r1.astype(jnp.bfloat16)
    c_lo = (r1 - c_mid.astype(jnp.float32)).astype(jnp.bfloat16)

    def oh_mm(c_part):
        return jax.lax.dot_general(
            onehot, c_part, (((1,), (0,)), ((), ())),
            preferred_element_type=jnp.float32)

    chosen = (oh_mm(c_hi) + oh_mm(c_mid)) + oh_mm(c_lo)
    return idx, chosen


def _mid_stage_kernel(res_ref, rn_ref, cn_ref, codes_ref, res_out_ref,
                      idx_ref):
    idx, chosen = _select_chosen(res_ref[...], rn_ref[...], cn_ref[...],
                                 codes_ref[...])
    res_out_ref[...] = res_ref[...] - chosen
    idx_ref[...] = idx[:, None]


def _last_stage_kernel(res_ref, rn_ref, cn_ref, codes_ref, x_ref, q_ref,
                       idx_ref):
    idx, chosen = _select_chosen(res_ref[...], rn_ref[...], cn_ref[...],
                                 codes_ref[...])
    q_ref[...] = x_ref[...] - (res_ref[...] - chosen)
    idx_ref[...] = idx[:, None]


def _row_spec():
    return pl.BlockSpec((M_BLK, D), lambda i: (i, 0))


def _fixed_specs():
    return [
        pl.BlockSpec((M_BLK, 1), lambda i: (i, 0)),
        pl.BlockSpec((1, K), lambda i: (0, 0)),
        pl.BlockSpec((K, D), lambda i: (0, 0)),
    ]


def _out_specs(n):
    return (
        [_row_spec(), pl.BlockSpec((M_BLK, 1), lambda i: (i, 0))],
        [jax.ShapeDtypeStruct((n, D), jnp.float32),
         jax.ShapeDtypeStruct((n, 1), jnp.int32)],
    )


def _run_mid_stage(residual, rn, cn, codes):
    n = residual.shape[0]
    out_specs, out_shape = _out_specs(n)
    return pl.pallas_call(
        _mid_stage_kernel,
        grid=(n // M_BLK,),
        in_specs=[_row_spec()] + _fixed_specs(),
        out_specs=out_specs,
        out_shape=out_shape,
    )(residual, rn, cn, codes)


def _run_last_stage(residual, rn, cn, codes, x_flat):
    n = residual.shape[0]
    out_specs, out_shape = _out_specs(n)
    return pl.pallas_call(
        _last_stage_kernel,
        grid=(n // M_BLK,),
        in_specs=[_row_spec()] + _fixed_specs() + [_row_spec()],
        out_specs=out_specs,
        out_shape=out_shape,
    )(residual, rn, cn, codes, x_flat)


def kernel(x_real, codebooks):
    b, t, d = x_real.shape
    n = b * t
    x_flat = x_real.reshape(n, d)
    cn_all = jnp.sum(codebooks ** 2, axis=2)
    residual = x_flat
    indices = []
    for s in range(N_STAGES):
        rn = jnp.sum(residual ** 2, axis=1, keepdims=True)
        cn = cn_all[s][None, :]
        if s < N_STAGES - 1:
            residual, idx = _run_mid_stage(residual, rn, cn, codebooks[s])
        else:
            q, idx = _run_last_stage(residual, rn, cn, codebooks[s], x_flat)
        indices.append(idx[:, 0].reshape(b, t))
    return q.reshape(b, t, d), jnp.stack(indices, axis=0)


# speedup vs baseline: 1.1190x; 1.1190x over previous
import jax
import jax.numpy as jnp
from jax.experimental import pallas as pl

N_STAGES = 4
K = 1024
D = 256
M_BLK = 1024


def _select_chosen(residual, rn, cn, codes):
    dots = jax.lax.dot_general(
        residual, codes, (((1,), (1,)), ((), ())),
        preferred_element_type=jnp.float32)
    dists = (rn + cn) - 2.0 * dots
    dmin = jnp.min(dists, axis=1, keepdims=True)
    col = jax.lax.broadcasted_iota(jnp.int32, dists.shape, 1)
    idx = jnp.min(jnp.where(dists == dmin, col, K), axis=1)
    onehot = (col == idx[:, None]).astype(jnp.float32)
    c_hi = codes.astype(jnp.bfloat16).astype(jnp.float32)
    r1 = codes - c_hi
    c_mid = r1.astype(jnp.bfloat16).astype(jnp.float32)
    c_lo = r1 - c_mid

    def oh_mm(c_part):
        return jax.lax.dot_general(
            onehot, c_part, (((1,), (0,)), ((), ())),
            preferred_element_type=jnp.float32)

    chosen = (oh_mm(c_hi) + oh_mm(c_mid)) + oh_mm(c_lo)
    return idx, chosen


def _mid_stage_kernel(res_ref, rn_ref, cn_ref, codes_ref, res_out_ref,
                      idx_ref):
    idx, chosen = _select_chosen(res_ref[...], rn_ref[...], cn_ref[...],
                                 codes_ref[...])
    res_out_ref[...] = res_ref[...] - chosen
    idx_ref[...] = idx[:, None]


def _last_stage_kernel(res_ref, rn_ref, cn_ref, codes_ref, x_ref, q_ref,
                       idx_ref):
    idx, chosen = _select_chosen(res_ref[...], rn_ref[...], cn_ref[...],
                                 codes_ref[...])
    q_ref[...] = x_ref[...] - (res_ref[...] - chosen)
    idx_ref[...] = idx[:, None]


def _row_spec():
    return pl.BlockSpec((M_BLK, D), lambda i: (i, 0))


def _fixed_specs():
    return [
        pl.BlockSpec((M_BLK, 1), lambda i: (i, 0)),
        pl.BlockSpec((1, K), lambda i: (0, 0)),
        pl.BlockSpec((K, D), lambda i: (0, 0)),
    ]


def _out_specs(n):
    return (
        [_row_spec(), pl.BlockSpec((M_BLK, 1), lambda i: (i, 0))],
        [jax.ShapeDtypeStruct((n, D), jnp.float32),
         jax.ShapeDtypeStruct((n, 1), jnp.int32)],
    )


def _run_mid_stage(residual, rn, cn, codes):
    n = residual.shape[0]
    out_specs, out_shape = _out_specs(n)
    return pl.pallas_call(
        _mid_stage_kernel,
        grid=(n // M_BLK,),
        in_specs=[_row_spec()] + _fixed_specs(),
        out_specs=out_specs,
        out_shape=out_shape,
    )(residual, rn, cn, codes)


def _run_last_stage(residual, rn, cn, codes, x_flat):
    n = residual.shape[0]
    out_specs, out_shape = _out_specs(n)
    return pl.pallas_call(
        _last_stage_kernel,
        grid=(n // M_BLK,),
        in_specs=[_row_spec()] + _fixed_specs() + [_row_spec()],
        out_specs=out_specs,
        out_shape=out_shape,
    )(residual, rn, cn, codes, x_flat)


def kernel(x_real, codebooks):
    b, t, d = x_real.shape
    n = b * t
    x_flat = x_real.reshape(n, d)
    cn_all = jnp.sum(codebooks ** 2, axis=2)
    residual = x_flat
    indices = []
    for s in range(N_STAGES):
        rn = jnp.sum(residual ** 2, axis=1, keepdims=True)
        cn = cn_all[s][None, :]
        if s < N_STAGES - 1:
            residual, idx = _run_mid_stage(residual, rn, cn, codebooks[s])
        else:
            q, idx = _run_last_stage(residual, rn, cn, codebooks[s], x_flat)
        indices.append(idx[:, 0].reshape(b, t))
    return q.reshape(b, t, d), jnp.stack(indices, axis=0)


# speedup vs baseline: 1.1662x; 1.0422x over previous
import jax
import jax.numpy as jnp
from jax.experimental import pallas as pl

N_STAGES = 4
K = 1024
D = 256
M_BLK = 2048


def _select_chosen(residual, rn, cn, codes):
    dots = jax.lax.dot_general(
        residual, codes, (((1,), (1,)), ((), ())),
        preferred_element_type=jnp.float32)
    dists = (rn + cn) - 2.0 * dots
    dmin = jnp.min(dists, axis=1, keepdims=True)
    col = jax.lax.broadcasted_iota(jnp.int32, dists.shape, 1)
    idx = jnp.min(jnp.where(dists == dmin, col, K), axis=1)
    onehot = (col == idx[:, None]).astype(jnp.float32)
    c_hi = codes.astype(jnp.bfloat16).astype(jnp.float32)
    r1 = codes - c_hi
    c_mid = r1.astype(jnp.bfloat16).astype(jnp.float32)
    c_lo = r1 - c_mid

    def oh_mm(c_part):
        return jax.lax.dot_general(
            onehot, c_part, (((1,), (0,)), ((), ())),
            preferred_element_type=jnp.float32)

    chosen = (oh_mm(c_hi) + oh_mm(c_mid)) + oh_mm(c_lo)
    return idx, chosen


def _mid_stage_kernel(res_ref, rn_ref, cn_ref, codes_ref, res_out_ref,
                      idx_ref):
    idx, chosen = _select_chosen(res_ref[...], rn_ref[...], cn_ref[...],
                                 codes_ref[...])
    res_out_ref[...] = res_ref[...] - chosen
    idx_ref[...] = idx[:, None]


def _last_stage_kernel(res_ref, rn_ref, cn_ref, codes_ref, x_ref, q_ref,
                       idx_ref):
    idx, chosen = _select_chosen(res_ref[...], rn_ref[...], cn_ref[...],
                                 codes_ref[...])
    q_ref[...] = x_ref[...] - (res_ref[...] - chosen)
    idx_ref[...] = idx[:, None]


def _row_spec():
    return pl.BlockSpec((M_BLK, D), lambda i: (i, 0))


def _fixed_specs():
    return [
        pl.BlockSpec((M_BLK, 1), lambda i: (i, 0)),
        pl.BlockSpec((1, K), lambda i: (0, 0)),
        pl.BlockSpec((K, D), lambda i: (0, 0)),
    ]


def _out_specs(n):
    return (
        [_row_spec(), pl.BlockSpec((M_BLK, 1), lambda i: (i, 0))],
        [jax.ShapeDtypeStruct((n, D), jnp.float32),
         jax.ShapeDtypeStruct((n, 1), jnp.int32)],
    )


def _run_mid_stage(residual, rn, cn, codes):
    n = residual.shape[0]
    out_specs, out_shape = _out_specs(n)
    return pl.pallas_call(
        _mid_stage_kernel,
        grid=(n // M_BLK,),
        in_specs=[_row_spec()] + _fixed_specs(),
        out_specs=out_specs,
        out_shape=out_shape,
    )(residual, rn, cn, codes)


def _run_last_stage(residual, rn, cn, codes, x_flat):
    n = residual.shape[0]
    out_specs, out_shape = _out_specs(n)
    return pl.pallas_call(
        _last_stage_kernel,
        grid=(n // M_BLK,),
        in_specs=[_row_spec()] + _fixed_specs() + [_row_spec()],
        out_specs=out_specs,
        out_shape=out_shape,
    )(residual, rn, cn, codes, x_flat)


def kernel(x_real, codebooks):
    b, t, d = x_real.shape
    n = b * t
    x_flat = x_real.reshape(n, d)
    cn_all = jnp.sum(codebooks ** 2, axis=2)
    residual = x_flat
    indices = []
    for s in range(N_STAGES):
        rn = jnp.sum(residual ** 2, axis=1, keepdims=True)
        cn = cn_all[s][None, :]
        if s < N_STAGES - 1:
            residual, idx = _run_mid_stage(residual, rn, cn, codebooks[s])
        else:
            q, idx = _run_last_stage(residual, rn, cn, codebooks[s], x_flat)
        indices.append(idx[:, 0].reshape(b, t))
    return q.reshape(b, t, d), jnp.stack(indices, axis=0)


# speedup vs baseline: 1.1708x; 1.0039x over previous
import jax
import jax.numpy as jnp
from jax.experimental import pallas as pl

N_STAGES = 4
K = 1024
D = 256
M_BLK = 3072


def _select_chosen(residual, rn, cn, codes):
    dots = jax.lax.dot_general(
        residual, codes, (((1,), (1,)), ((), ())),
        preferred_element_type=jnp.float32)
    dists = (rn + cn) - 2.0 * dots
    dmin = jnp.min(dists, axis=1, keepdims=True)
    col = jax.lax.broadcasted_iota(jnp.int32, dists.shape, 1)
    idx = jnp.min(jnp.where(dists == dmin, col, K), axis=1)
    onehot = (col == idx[:, None]).astype(jnp.float32)
    c_hi = codes.astype(jnp.bfloat16).astype(jnp.float32)
    r1 = codes - c_hi
    c_mid = r1.astype(jnp.bfloat16).astype(jnp.float32)
    c_lo = r1 - c_mid

    def oh_mm(c_part):
        return jax.lax.dot_general(
            onehot, c_part, (((1,), (0,)), ((), ())),
            preferred_element_type=jnp.float32)

    chosen = (oh_mm(c_hi) + oh_mm(c_mid)) + oh_mm(c_lo)
    return idx, chosen


def _mid_stage_kernel(res_ref, rn_ref, cn_ref, codes_ref, res_out_ref,
                      idx_ref):
    idx, chosen = _select_chosen(res_ref[...], rn_ref[...], cn_ref[...],
                                 codes_ref[...])
    res_out_ref[...] = res_ref[...] - chosen
    idx_ref[...] = idx[:, None]


def _last_stage_kernel(res_ref, rn_ref, cn_ref, codes_ref, x_ref, q_ref,
                       idx_ref):
    idx, chosen = _select_chosen(res_ref[...], rn_ref[...], cn_ref[...],
                                 codes_ref[...])
    q_ref[...] = x_ref[...] - (res_ref[...] - chosen)
    idx_ref[...] = idx[:, None]


def _row_spec():
    return pl.BlockSpec((M_BLK, D), lambda i: (i, 0))


def _fixed_specs():
    return [
        pl.BlockSpec((M_BLK, 1), lambda i: (i, 0)),
        pl.BlockSpec((1, K), lambda i: (0, 0)),
        pl.BlockSpec((K, D), lambda i: (0, 0)),
    ]


def _out_specs(n):
    return (
        [_row_spec(), pl.BlockSpec((M_BLK, 1), lambda i: (i, 0))],
        [jax.ShapeDtypeStruct((n, D), jnp.float32),
         jax.ShapeDtypeStruct((n, 1), jnp.int32)],
    )


def _run_mid_stage(residual, rn, cn, codes):
    n = residual.shape[0]
    out_specs, out_shape = _out_specs(n)
    return pl.pallas_call(
        _mid_stage_kernel,
        grid=(n // M_BLK,),
        in_specs=[_row_spec()] + _fixed_specs(),
        out_specs=out_specs,
        out_shape=out_shape,
    )(residual, rn, cn, codes)


def _run_last_stage(residual, rn, cn, codes, x_flat):
    n = residual.shape[0]
    out_specs, out_shape = _out_specs(n)
    return pl.pallas_call(
        _last_stage_kernel,
        grid=(n // M_BLK,),
        in_specs=[_row_spec()] + _fixed_specs() + [_row_spec()],
        out_specs=out_specs,
        out_shape=out_shape,
    )(residual, rn, cn, codes, x_flat)


def kernel(x_real, codebooks):
    b, t, d = x_real.shape
    n = b * t
    x_flat = x_real.reshape(n, d)
    cn_all = jnp.sum(codebooks ** 2, axis=2)
    residual = x_flat
    indices = []
    for s in range(N_STAGES):
        rn = jnp.sum(residual ** 2, axis=1, keepdims=True)
        cn = cn_all[s][None, :]
        if s < N_STAGES - 1:
            residual, idx = _run_mid_stage(residual, rn, cn, codebooks[s])
        else:
            q, idx = _run_last_stage(residual, rn, cn, codebooks[s], x_flat)
        indices.append(idx[:, 0].reshape(b, t))
    return q.reshape(b, t, d), jnp.stack(indices, axis=0)
